# Initial kernel scaffold; baseline (speedup 1.0000x reference)
import jax
import jax.numpy as jnp
from jax.experimental import pallas as pl


def kernel(x, edge_index, W1, att_src1, att_dst1, b1, W2, att_src2, att_dst2, b2):
    raise NotImplementedError("write your pallas kernel here")


# speedup vs baseline: 5.4357x; 5.4357x over previous
import functools

import jax
import jax.numpy as jnp
from jax import lax
from jax.experimental import pallas as pl
from jax.experimental.pallas import tpu as pltpu
from jax.experimental.pallas import tpu_sc as plsc

N = 10000
E = 320000
F_IN = 128
HID = 128
H1 = 8

NC = 2
NS = 16
L = 16
NW = NC * NS
NP = 10240
RT = 320
PR = NP // 16
EP = 327680
BA = 256
BE = 128
SCH = 32

f32 = jnp.float32
i32 = jnp.int32


def _iota():
    return lax.iota(i32, L)


def _splat(v):
    return jnp.full((L,), v, i32)


def _al8(v):
    return pl.multiple_of(v, 8)


def _sca_body(asp, adp, ms16, src, dst,
              ex, dinit,
              idxs, idxd, gbuf, sa, exo, msb, sem):
    cid = lax.axis_index("c")
    sid = lax.axis_index("s")
    wid = cid * NS + sid

    pltpu.sync_copy(ms16, msb)
    msv = msb[0, pl.ds(0, L)]
    lane = _iota()
    lo = lane < 8

    def edge_ex(w_s, w_d):
        z = w_s + w_d
        alpha = jnp.maximum(z, 0.2 * z)
        kk = jnp.maximum(w_d + msv, 0.0)
        return jnp.exp(alpha - kk)

    rc0 = sid * 2 * RT

    @pl.when(cid == 0)
    def _():
        def chunk(c, _):
            rc = _al8(rc0 + 80 * c)
            pltpu.sync_copy(asp.at[pl.ds(rc, 80)], gbuf.at[pl.ds(0, 80)])
            pltpu.sync_copy(adp.at[pl.ds(rc, 80)], gbuf.at[pl.ds(80, 80)])

            def selfb(j, _):
                v0 = edge_ex(gbuf[2 * j, pl.ds(0, L)],
                             gbuf[80 + 2 * j, pl.ds(0, L)])
                v1 = edge_ex(gbuf[2 * j + 1, pl.ds(0, L)],
                             gbuf[80 + 2 * j + 1, pl.ds(0, L)])
                exo[j, pl.ds(0, L)] = jnp.where(lo, v0, v1)
                return 0
            lax.fori_loop(0, 40, selfb, 0)
            pltpu.sync_copy(exo.at[pl.ds(0, 40)],
                            dinit.at[pl.ds(_al8(sid * 320 + 40 * c), 40)])
            return 0
        lax.fori_loop(0, 8, chunk, 0)

    ebase = wid * (EP // NW)

    def batch(b, _):
        base = _al8(ebase + b * BA)
        pltpu.sync_copy(src.at[pl.ds(base, BA)], idxs)
        pltpu.sync_copy(dst.at[pl.ds(base, BA)], idxd)

        pltpu.async_copy(asp.at[idxs], gbuf, sem).wait()

        def extract(j, _):
            w0 = gbuf[2 * j, pl.ds(0, L)]
            w1 = gbuf[2 * j + 1, pl.ds(0, L)]
            sa[j, pl.ds(0, L)] = jnp.where(lo, w0, w1)
            return 0
        lax.fori_loop(0, BA // 2, extract, 0)

        pltpu.async_copy(adp.at[idxd], gbuf, sem).wait()

        def compute(j, _):
            w0 = gbuf[2 * j, pl.ds(0, L)]
            w1 = gbuf[2 * j + 1, pl.ds(0, L)]
            d = jnp.where(lo, w0, w1)
            exo[j, pl.ds(0, L)] = edge_ex(sa[j, pl.ds(0, L)], d)
            return 0
        lax.fori_loop(0, BA // 2, compute, 0)

        pltpu.sync_copy(exo, ex.at[pl.ds(_al8(base // 2), BA * 8 // L)])
        return 0

    lax.fori_loop(0, EP // NW // BA, batch, 0)


_sca = functools.partial(
    pl.kernel,
    _sca_body,
    out_type=(
        jax.ShapeDtypeStruct((EP * 8 // L, L), f32),
        jax.ShapeDtypeStruct((NP * 8 // L, L), f32),
    ),
    mesh=plsc.VectorSubcoreMesh(core_axis_name="c", subcore_axis_name="s"),
    scratch_types=[
        pltpu.VMEM((BA,), i32),
        pltpu.VMEM((BA,), i32),
        pltpu.VMEM((BA, 128), f32),
        pltpu.VMEM((BA * 8 // L, L), f32),
        pltpu.VMEM((BA * 8 // L, L), f32),
        pltpu.VMEM((1, L), f32),
        pltpu.SemaphoreType.DMA,
    ],
)()


def _row_scale(buf, r, s):
    for k in range(8):
        v = buf[r, pl.ds(k * L, L)]
        buf[r, pl.ds(k * L, L)] = v * s


def _make_msg(nht):
    hps = 4 if nht == 8 else 1
    et = EP // NS if nht == 8 else EP // NW
    nb = et // BE
    out_rows = 8 * NP if nht == 8 else 2 * NP

    def body(table, idxarr, exarr, dinit, dst, out, denomp,
             srcb, dstb, exb, rbuf, dbuf, acc, sem):
        cid = lax.axis_index("c")
        sid = lax.axis_index("s")
        wid = cid * NS + sid
        lane = _iota()
        lane8 = lane % 8
        lane8p = lane8 + 8
        ebase = sid * et if nht == 8 else wid * et

        def acc_chunks(fill, writeback_to, obase):
            def chunk(c, _):
                rc = _al8(sid * 2 * RT + c * 80)
                fill(c, rc)
                pltpu.sync_copy(rbuf.at[pl.ds(0, 80)], acc.at[pl.ds(rc, 80)])
                return 0
            lax.fori_loop(0, 8, chunk, 0)

        def writeback(dest, obase):
            def wb(c, _):
                rc = _al8(sid * 2 * RT + c * 80)
                pltpu.sync_copy(acc.at[pl.ds(rc, 80)], rbuf.at[pl.ds(0, 80)])
                pltpu.sync_copy(rbuf.at[pl.ds(0, 80)],
                                dest.at[pl.ds(_al8(obase + rc), 80)])
                return 0
            lax.fori_loop(0, 8, wb, 0)

        for k in range(hps):
            hidx = cid * hps + k if nht == 8 else 0

            def init_msg():
                def fill(c, rc):
                    pltpu.sync_copy(table.at[pl.ds(_al8(hidx * NP + rc), 80)],
                                    rbuf.at[pl.ds(0, 80)])
                    pltpu.sync_copy(dinit.at[pl.ds(_al8(rc // 2), 40)], dbuf)

                    def scale(r, _):
                        fl = r * 8 + hidx
                        w = dbuf[fl // L, pl.ds(0, L)]
                        s = w[jnp.full((L,), fl % L, i32)]
                        _row_scale(rbuf, r, s)
                        return 0
                    lax.fori_loop(0, 80, scale, 0)
                acc_chunks(fill, None, 0)

            def init_zero():
                def zro(r, _):
                    zv = jnp.zeros((L,), f32)
                    for kk in range(8):
                        rbuf[r, pl.ds(kk * L, L)] = zv
                    return 0
                lax.fori_loop(0, 80, zro, 0)

                def fill(c, rc):
                    pass
                acc_chunks(fill, None, 0)

            if nht == 8:
                init_msg()
            else:
                @pl.when(cid == 0)
                def _():
                    init_msg()

                @pl.when(cid == 1)
                def _():
                    init_zero()

            plsc.subcore_barrier()

            def batch(b, _):
                base = _al8(ebase + b * BE)
                pltpu.sync_copy(idxarr.at[pl.ds(_al8(hidx * EP + base), BE)],
                                srcb)
                pltpu.sync_copy(exarr.at[pl.ds(_al8(base // 2), BE * 8 // L)],
                                exb)
                pltpu.async_copy(table.at[srcb], rbuf, sem).wait()

                def scale(e, _):
                    fl = e * 8 + hidx
                    w = exb[fl // L, pl.ds(0, L)]
                    s = w[jnp.full((L,), fl % L, i32)]
                    _row_scale(rbuf, e, s)
                    return 0
                lax.fori_loop(0, BE, scale, 0)

                def scat(k5, _):
                    pltpu.sync_copy(dst.at[pl.ds(_al8(base + k5 * SCH), SCH)],
                                    dstb)
                    pltpu.sync_copy(rbuf.at[pl.ds(_al8(k5 * SCH), SCH)],
                                    acc.at[dstb], add=True)
                    return 0
                lax.fori_loop(0, BE // SCH, scat, 0)
                return 0

            lax.fori_loop(0, nb, batch, 0)
            plsc.subcore_barrier()
            writeback(out, hidx * NP if nht == 8 else cid * NP)
            plsc.subcore_barrier()

        def init_den():
            def fill(c, rc):
                pltpu.sync_copy(dinit.at[pl.ds(_al8(rc // 2), 40)], dbuf)

                def bld(j, _):
                    w = dbuf[j, pl.ds(0, L)]
                    r0 = w[lane8]
                    r1 = w[lane8p]
                    for kk in range(8):
                        rbuf[2 * j, pl.ds(kk * L, L)] = r0
                        rbuf[2 * j + 1, pl.ds(kk * L, L)] = r1
                    return 0
                lax.fori_loop(0, 40, bld, 0)
            acc_chunks(fill, None, 0)

        def init_zero2():
            def zro(r, _):
                zv = jnp.zeros((L,), f32)
                for kk in range(8):
                    rbuf[r, pl.ds(kk * L, L)] = zv
                return 0
            lax.fori_loop(0, 80, zro, 0)

            def fill(c, rc):
                pass
            acc_chunks(fill, None, 0)

        if nht == 8:
            init_den()
        else:
            @pl.when(cid == 0)
            def _():
                init_den()

            @pl.when(cid == 1)
            def _():
                init_zero2()

        plsc.subcore_barrier()

        def dbatch(b, _):
            base = _al8(ebase + b * BE)
            pltpu.sync_copy(exarr.at[pl.ds(_al8(base // 2), BE * 8 // L)],
                            exb)

            def bld(j, _):
                v = exb[j, pl.ds(0, L)]
                r0 = v[lane8]
                r1 = v[lane8p]
                for kk in range(8):
                    rbuf[2 * j, pl.ds(kk * L, L)] = r0
                    rbuf[2 * j + 1, pl.ds(kk * L, L)] = r1
                return 0
            lax.fori_loop(0, BE // 2, bld, 0)

            def scat(k5, _):
                pltpu.sync_copy(dst.at[pl.ds(_al8(base + k5 * SCH), SCH)],
                                dstb)
                pltpu.sync_copy(rbuf.at[pl.ds(_al8(k5 * SCH), SCH)],
                                acc.at[dstb], add=True)
                return 0
            lax.fori_loop(0, BE // SCH, scat, 0)
            return 0

        lax.fori_loop(0, nb, dbatch, 0)
        plsc.subcore_barrier()
        writeback(denomp, cid * NP)

    return functools.partial(
        pl.kernel,
        body,
        out_type=(
            jax.ShapeDtypeStruct((out_rows, 128), f32),
            jax.ShapeDtypeStruct((2 * NP, 128), f32),
        ),
        mesh=plsc.VectorSubcoreMesh(core_axis_name="c",
                                    subcore_axis_name="s"),
        scratch_types=[
            pltpu.VMEM((BE,), i32),
            pltpu.VMEM((SCH,), i32),
            pltpu.VMEM((BE * 8 // L, L), f32),
            pltpu.VMEM((BE, 128), f32),
            pltpu.VMEM((40, L), f32),
            pltpu.VMEM_SHARED((NP, 128), f32),
            pltpu.SemaphoreType.DMA,
        ],
    )()


_msg1 = _make_msg(8)
_msg2 = _make_msg(1)


B1BLK = 1024


def _tc1_body(x_ref, w_ref, as_ref, ad_ref, h_ref, asrc_ref, adst_ref,
              ms_ref):
    h = jnp.dot(x_ref[...], w_ref[...], preferred_element_type=f32)
    h3 = h.reshape(B1BLK, H1, HID)
    a_s = jnp.sum(h3 * as_ref[...][None], axis=-1)
    a_d = jnp.sum(h3 * ad_ref[...][None], axis=-1)
    h_ref[...] = h3.transpose(1, 0, 2)
    asrc_ref[...] = jnp.broadcast_to(a_s[:, None, :],
                                     (B1BLK, 16, H1)).reshape(B1BLK, 128)
    adst_ref[...] = jnp.broadcast_to(a_d[:, None, :],
                                     (B1BLK, 16, H1)).reshape(B1BLK, 128)
    m = jnp.broadcast_to(jnp.max(a_s, axis=0)[:, None], (H1, HID))
    @pl.when(pl.program_id(0) == 0)
    def _():
        ms_ref[...] = m
    @pl.when(pl.program_id(0) > 0)
    def _():
        ms_ref[...] = jnp.maximum(ms_ref[...], m)


def _tc1(x_p, W1, as1, ad1):
    return pl.pallas_call(
        _tc1_body,
        grid=(NP // B1BLK,),
        in_specs=[
            pl.BlockSpec((B1BLK, F_IN), lambda i: (i, 0)),
            pl.BlockSpec((F_IN, H1 * HID), lambda i: (0, 0)),
            pl.BlockSpec((H1, HID), lambda i: (0, 0)),
            pl.BlockSpec((H1, HID), lambda i: (0, 0)),
        ],
        out_specs=[
            pl.BlockSpec((H1, B1BLK, HID), lambda i: (0, i, 0)),
            pl.BlockSpec((B1BLK, 128), lambda i: (i, 0)),
            pl.BlockSpec((B1BLK, 128), lambda i: (i, 0)),
            pl.BlockSpec((H1, HID), lambda i: (0, 0)),
        ],
        out_shape=[
            jax.ShapeDtypeStruct((H1, NP, HID), f32),
            jax.ShapeDtypeStruct((NP, 128), f32),
            jax.ShapeDtypeStruct((NP, 128), f32),
            jax.ShapeDtypeStruct((H1, HID), f32),
        ],
    )(x_p, W1, as1, ad1)


B2BLK = 2048


def _tc2_body(acc_ref, den_ref, b1_ref, w2_ref, as2_ref, ad2_ref,
              h2_ref, asrc_ref, adst_ref, ms_ref):
    den = den_ref[0][:, 0:8]
    hcat = acc_ref[...].transpose(1, 0, 2)
    o = hcat / den[:, :, None] + b1_ref[...][None]
    o = jnp.where(o > 0, o, jnp.exp(jnp.minimum(o, 0.0)) - 1.0)
    hl = o.reshape(B2BLK, H1 * HID)
    h2 = jnp.dot(hl, w2_ref[...], preferred_element_type=f32)
    a_s = jnp.sum(h2 * as2_ref[...], axis=-1, keepdims=True)
    a_d = jnp.sum(h2 * ad2_ref[...], axis=-1, keepdims=True)
    h2_ref[...] = h2
    asrc_ref[...] = jnp.broadcast_to(a_s, (B2BLK, 128))
    adst_ref[...] = jnp.broadcast_to(a_d, (B2BLK, 128))
    m = jnp.broadcast_to(jnp.max(a_s), (H1, HID))
    @pl.when(pl.program_id(0) == 0)
    def _():
        ms_ref[...] = m
    @pl.when(pl.program_id(0) > 0)
    def _():
        ms_ref[...] = jnp.maximum(ms_ref[...], m)


def _tc2(acc1, denp1, b1r, W2, as2, ad2):
    return pl.pallas_call(
        _tc2_body,
        grid=(NP // B2BLK,),
        in_specs=[
            pl.BlockSpec((H1, B2BLK, HID), lambda i: (0, i, 0)),
            pl.BlockSpec((1, B2BLK, 128), lambda i: (0, i, 0)),
            pl.BlockSpec((H1, HID), lambda i: (0, 0)),
            pl.BlockSpec((H1 * HID, HID), lambda i: (0, 0)),
            pl.BlockSpec((1, HID), lambda i: (0, 0)),
            pl.BlockSpec((1, HID), lambda i: (0, 0)),
        ],
        out_specs=[
            pl.BlockSpec((B2BLK, HID), lambda i: (i, 0)),
            pl.BlockSpec((B2BLK, 128), lambda i: (i, 0)),
            pl.BlockSpec((B2BLK, 128), lambda i: (i, 0)),
            pl.BlockSpec((H1, HID), lambda i: (0, 0)),
        ],
        out_shape=[
            jax.ShapeDtypeStruct((NP, HID), f32),
            jax.ShapeDtypeStruct((NP, 128), f32),
            jax.ShapeDtypeStruct((NP, 128), f32),
            jax.ShapeDtypeStruct((H1, HID), f32),
        ],
    )(acc1, denp1, b1r, W2, as2, ad2)


B3BLK = 2000


def _tc3_body(acc_ref, den_ref, b2_ref, out_ref):
    den = den_ref[0][:, 0:1] + den_ref[1][:, 0:1]
    out_ref[...] = (acc_ref[0] + acc_ref[1]) / den + b2_ref[...]


def _tc3(acc2, denp2, b2r):
    return pl.pallas_call(
        _tc3_body,
        grid=(N // B3BLK,),
        in_specs=[
            pl.BlockSpec((2, B3BLK, HID), lambda i: (0, i, 0)),
            pl.BlockSpec((2, B3BLK, 128), lambda i: (0, i, 0)),
            pl.BlockSpec((1, HID), lambda i: (0, 0)),
        ],
        out_specs=pl.BlockSpec((B3BLK, HID), lambda i: (i, 0)),
        out_shape=jax.ShapeDtypeStruct((N, HID), f32),
    )(acc2, denp2, b2r)


def kernel(x, edge_index, W1, att_src1, att_dst1, b1,
           W2, att_src2, att_dst2, b2):
    pad = jnp.full((EP - E,), NP - 1, dtype=i32)
    src = jnp.concatenate([edge_index[0].astype(i32), pad])
    dst = jnp.concatenate([edge_index[1].astype(i32), pad])
    idx1 = (jnp.arange(H1, dtype=i32)[:, None] * NP
            + src[None, :]).reshape(-1)
    x_p = jnp.pad(x, ((0, NP - N), (0, 0)))

    h1, asrc1, adst1, ms1 = _tc1(x_p, W1,
                                 att_src1.reshape(H1, HID),
                                 att_dst1.reshape(H1, HID))
    ms16_1 = jnp.tile(ms1[:, 0], 2).reshape(1, L)

    ex1, dinit1 = _sca(asrc1, adst1, ms16_1, src, dst)
    acc1, denp1 = _msg1(h1.reshape(H1 * NP, HID), idx1, ex1, dinit1, dst)

    h2, asrc2, adst2, ms2 = _tc2(
        acc1.reshape(H1, NP, HID),
        denp1.reshape(2, NP, 128),
        b1.reshape(H1, HID), W2,
        att_src2.reshape(1, HID), att_dst2.reshape(1, HID))
    ms16_2 = jnp.broadcast_to(ms2[0, 0], (1, L))

    ex2, dinit2 = _sca(asrc2, adst2, ms16_2, src, dst)
    acc2, denp2 = _msg2(h2, src, ex2, dinit2, dst)

    return _tc3(acc2.reshape(2, NP, HID), denp2.reshape(2, NP, 128),
                b2.reshape(1, HID))


# speedup vs baseline: 5.6066x; 1.0315x over previous
import functools

import jax
import jax.numpy as jnp
from jax import lax
from jax.experimental import pallas as pl
from jax.experimental.pallas import tpu as pltpu
from jax.experimental.pallas import tpu_sc as plsc

N = 10000
E = 320000
F_IN = 128
HID = 128
H1 = 8

NC = 2
NS = 16
L = 16
NW = NC * NS
NP = 10240
RT = 320
PR = NP // 16
EP = 327680
BA = 256
BE = 128
SCH = 32

f32 = jnp.float32
i32 = jnp.int32


def _iota():
    return lax.iota(i32, L)


def _splat(v):
    return jnp.full((L,), v, i32)


def _al8(v):
    return pl.multiple_of(v, 8)


def _sca_body(asp, adp, ms16, src, dst,
              ex, dinit,
              idxs, idxd, gbuf, sa, exo, msb, sem):
    cid = lax.axis_index("c")
    sid = lax.axis_index("s")
    wid = cid * NS + sid

    pltpu.sync_copy(ms16, msb)
    msv = msb[0, pl.ds(0, L)]
    lane = _iota()
    lo = lane < 8

    def edge_ex(w_s, w_d):
        z = w_s + w_d
        alpha = jnp.maximum(z, 0.2 * z)
        kk = jnp.maximum(w_d + msv, 0.0)
        return jnp.exp(alpha - kk)

    rc0 = sid * 2 * RT

    def chunk(c, _):
        rc = _al8(rc0 + 80 * c)
        pltpu.sync_copy(asp.at[pl.ds(rc, 80)], gbuf.at[pl.ds(0, 80)])
        pltpu.sync_copy(adp.at[pl.ds(rc, 80)], gbuf.at[pl.ds(80, 80)])

        def selfb(j, _):
            v0 = edge_ex(gbuf[2 * j, pl.ds(0, L)],
                         gbuf[80 + 2 * j, pl.ds(0, L)])
            v1 = edge_ex(gbuf[2 * j + 1, pl.ds(0, L)],
                         gbuf[80 + 2 * j + 1, pl.ds(0, L)])
            exo[j, pl.ds(0, L)] = jnp.where(lo, v0, v1)
            return 0
        lax.fori_loop(0, 40, selfb, 0)
        pltpu.sync_copy(exo.at[pl.ds(0, 40)],
                        dinit.at[pl.ds(_al8(sid * 320 + 40 * c), 40)])
        return 0
    lax.fori_loop(cid * 4, cid * 4 + 4, chunk, 0)

    ebase = wid * (EP // NW)

    def batch(b, _):
        base = _al8(ebase + b * BA)
        pltpu.sync_copy(src.at[pl.ds(base, BA)], idxs)
        pltpu.sync_copy(dst.at[pl.ds(base, BA)], idxd)

        pltpu.async_copy(asp.at[idxs], gbuf, sem).wait()

        def extract(j, _):
            w0 = gbuf[2 * j, pl.ds(0, L)]
            w1 = gbuf[2 * j + 1, pl.ds(0, L)]
            sa[j, pl.ds(0, L)] = jnp.where(lo, w0, w1)
            return 0
        lax.fori_loop(0, BA // 2, extract, 0)

        pltpu.async_copy(adp.at[idxd], gbuf, sem).wait()

        def compute(j, _):
            w0 = gbuf[2 * j, pl.ds(0, L)]
            w1 = gbuf[2 * j + 1, pl.ds(0, L)]
            d = jnp.where(lo, w0, w1)
            exo[j, pl.ds(0, L)] = edge_ex(sa[j, pl.ds(0, L)], d)
            return 0
        lax.fori_loop(0, BA // 2, compute, 0)

        pltpu.sync_copy(exo, ex.at[pl.ds(_al8(base // 2), BA * 8 // L)])
        return 0

    lax.fori_loop(0, EP // NW // BA, batch, 0)


_sca = functools.partial(
    pl.kernel,
    _sca_body,
    out_type=(
        jax.ShapeDtypeStruct((EP * 8 // L, L), f32),
        jax.ShapeDtypeStruct((NP * 8 // L, L), f32),
    ),
    mesh=plsc.VectorSubcoreMesh(core_axis_name="c", subcore_axis_name="s"),
    scratch_types=[
        pltpu.VMEM((BA,), i32),
        pltpu.VMEM((BA,), i32),
        pltpu.VMEM((BA, 128), f32),
        pltpu.VMEM((BA * 8 // L, L), f32),
        pltpu.VMEM((BA * 8 // L, L), f32),
        pltpu.VMEM((1, L), f32),
        pltpu.SemaphoreType.DMA,
    ],
)()


def _row_scale(buf, r, s):
    for k in range(8):
        v = buf[r, pl.ds(k * L, L)]
        buf[r, pl.ds(k * L, L)] = v * s


def _make_msg(nht):
    hps = 4 if nht == 8 else 1
    et = EP // NS if nht == 8 else EP // NW
    nb = et // BE
    out_rows = 8 * NP if nht == 8 else 2 * NP

    def body(table, idxarr, exarr, dinit, dst, out, denomp,
             srcb, dstb, exb, rbuf, dbuf, acc, sem):
        cid = lax.axis_index("c")
        sid = lax.axis_index("s")
        wid = cid * NS + sid
        lane = _iota()
        lane8 = lane % 8
        lane8p = lane8 + 8
        ebase = sid * et if nht == 8 else wid * et
        det = EP // NW
        dbase = wid * det
        dnb = det // BE
        fb = cid * 4
        zb = (1 - cid) * 4

        def acc_range(fill, c0, c1):
            def chunk(c, _):
                rc = _al8(sid * 2 * RT + c * 80)
                fill(c, rc)
                pltpu.sync_copy(rbuf.at[pl.ds(0, 80)], acc.at[pl.ds(rc, 80)])
                return 0
            lax.fori_loop(c0, c1, chunk, 0)

        def zero_rbuf():
            def zro(r, _):
                zv = jnp.zeros((L,), f32)
                for kk in range(8):
                    rbuf[r, pl.ds(kk * L, L)] = zv
                return 0
            lax.fori_loop(0, 80, zro, 0)

        def nofill(c, rc):
            pass

        def writeback(dest, obase):
            def wb(c, _):
                rc = _al8(sid * 2 * RT + c * 80)
                pltpu.sync_copy(acc.at[pl.ds(rc, 80)], rbuf.at[pl.ds(0, 80)])
                pltpu.sync_copy(rbuf.at[pl.ds(0, 80)],
                                dest.at[pl.ds(_al8(obase + rc), 80)])
                return 0
            lax.fori_loop(0, 8, wb, 0)

        for k in range(hps):
            hidx = cid * hps + k if nht == 8 else 0

            def fill_msg(c, rc):
                pltpu.sync_copy(table.at[pl.ds(_al8(hidx * NP + rc), 80)],
                                rbuf.at[pl.ds(0, 80)])
                pltpu.sync_copy(dinit.at[pl.ds(_al8(rc // 2), 40)], dbuf)

                def scale(r, _):
                    fl = r * 8 + hidx
                    w = dbuf[fl // L, pl.ds(0, L)]
                    s = w[jnp.full((L,), fl % L, i32)]
                    _row_scale(rbuf, r, s)
                    return 0
                lax.fori_loop(0, 80, scale, 0)

            if nht == 8:
                acc_range(fill_msg, 0, 8)
            else:
                zero_rbuf()
                acc_range(nofill, zb, zb + 4)
                acc_range(fill_msg, fb, fb + 4)

            plsc.subcore_barrier()

            def batch(b, _):
                base = _al8(ebase + b * BE)
                pltpu.sync_copy(idxarr.at[pl.ds(_al8(hidx * EP + base), BE)],
                                srcb)
                pltpu.sync_copy(exarr.at[pl.ds(_al8(base // 2), BE * 8 // L)],
                                exb)
                pltpu.async_copy(table.at[srcb], rbuf, sem).wait()

                def scale(e, _):
                    fl = e * 8 + hidx
                    w = exb[fl // L, pl.ds(0, L)]
                    s = w[jnp.full((L,), fl % L, i32)]
                    _row_scale(rbuf, e, s)
                    return 0
                lax.fori_loop(0, BE, scale, 0)

                def scat(k5, _):
                    pltpu.sync_copy(dst.at[pl.ds(_al8(base + k5 * SCH), SCH)],
                                    dstb)
                    pltpu.sync_copy(rbuf.at[pl.ds(_al8(k5 * SCH), SCH)],
                                    acc.at[dstb], add=True)
                    return 0
                lax.fori_loop(0, BE // SCH, scat, 0)
                return 0

            lax.fori_loop(0, nb, batch, 0)
            plsc.subcore_barrier()
            writeback(out, hidx * NP if nht == 8 else cid * NP)
            plsc.subcore_barrier()

        def fill_den(c, rc):
            pltpu.sync_copy(dinit.at[pl.ds(_al8(rc // 2), 40)], dbuf)

            def bld(j, _):
                w = dbuf[j, pl.ds(0, L)]
                r0 = w[lane8]
                r1 = w[lane8p]
                for kk in range(8):
                    rbuf[2 * j, pl.ds(kk * L, L)] = r0
                    rbuf[2 * j + 1, pl.ds(kk * L, L)] = r1
                return 0
            lax.fori_loop(0, 40, bld, 0)

        zero_rbuf()
        acc_range(nofill, zb, zb + 4)
        acc_range(fill_den, fb, fb + 4)

        plsc.subcore_barrier()

        def dbatch(b, _):
            base = _al8(dbase + b * BE)
            pltpu.sync_copy(exarr.at[pl.ds(_al8(base // 2), BE * 8 // L)],
                            exb)

            def bld(j, _):
                v = exb[j, pl.ds(0, L)]
                r0 = v[lane8]
                r1 = v[lane8p]
                for kk in range(8):
                    rbuf[2 * j, pl.ds(kk * L, L)] = r0
                    rbuf[2 * j + 1, pl.ds(kk * L, L)] = r1
                return 0
            lax.fori_loop(0, BE // 2, bld, 0)

            def scat(k5, _):
                pltpu.sync_copy(dst.at[pl.ds(_al8(base + k5 * SCH), SCH)],
                                dstb)
                pltpu.sync_copy(rbuf.at[pl.ds(_al8(k5 * SCH), SCH)],
                                acc.at[dstb], add=True)
                return 0
            lax.fori_loop(0, BE // SCH, scat, 0)
            return 0

        lax.fori_loop(0, dnb, dbatch, 0)
        plsc.subcore_barrier()
        writeback(denomp, cid * NP)

    return functools.partial(
        pl.kernel,
        body,
        out_type=(
            jax.ShapeDtypeStruct((out_rows, 128), f32),
            jax.ShapeDtypeStruct((2 * NP, 128), f32),
        ),
        mesh=plsc.VectorSubcoreMesh(core_axis_name="c",
                                    subcore_axis_name="s"),
        scratch_types=[
            pltpu.VMEM((BE,), i32),
            pltpu.VMEM((SCH,), i32),
            pltpu.VMEM((BE * 8 // L, L), f32),
            pltpu.VMEM((BE, 128), f32),
            pltpu.VMEM((40, L), f32),
            pltpu.VMEM_SHARED((NP, 128), f32),
            pltpu.SemaphoreType.DMA,
        ],
    )()


_msg1 = _make_msg(8)
_msg2 = _make_msg(1)


B1BLK = 1024


def _tc1_body(x_ref, w_ref, as_ref, ad_ref, h_ref, asrc_ref, adst_ref,
              ms_ref):
    h = jnp.dot(x_ref[...], w_ref[...], preferred_element_type=f32)
    h3 = h.reshape(B1BLK, H1, HID)
    a_s = jnp.sum(h3 * as_ref[...][None], axis=-1)
    a_d = jnp.sum(h3 * ad_ref[...][None], axis=-1)
    h_ref[...] = h3.transpose(1, 0, 2)
    asrc_ref[...] = jnp.broadcast_to(a_s[:, None, :],
                                     (B1BLK, 16, H1)).reshape(B1BLK, 128)
    adst_ref[...] = jnp.broadcast_to(a_d[:, None, :],
                                     (B1BLK, 16, H1)).reshape(B1BLK, 128)
    m = jnp.broadcast_to(jnp.max(a_s, axis=0)[:, None], (H1, HID))
    @pl.when(pl.program_id(0) == 0)
    def _():
        ms_ref[...] = m
    @pl.when(pl.program_id(0) > 0)
    def _():
        ms_ref[...] = jnp.maximum(ms_ref[...], m)


def _tc1(x_p, W1, as1, ad1):
    return pl.pallas_call(
        _tc1_body,
        grid=(NP // B1BLK,),
        in_specs=[
            pl.BlockSpec((B1BLK, F_IN), lambda i: (i, 0)),
            pl.BlockSpec((F_IN, H1 * HID), lambda i: (0, 0)),
            pl.BlockSpec((H1, HID), lambda i: (0, 0)),
            pl.BlockSpec((H1, HID), lambda i: (0, 0)),
        ],
        out_specs=[
            pl.BlockSpec((H1, B1BLK, HID), lambda i: (0, i, 0)),
            pl.BlockSpec((B1BLK, 128), lambda i: (i, 0)),
            pl.BlockSpec((B1BLK, 128), lambda i: (i, 0)),
            pl.BlockSpec((H1, HID), lambda i: (0, 0)),
        ],
        out_shape=[
            jax.ShapeDtypeStruct((H1, NP, HID), f32),
            jax.ShapeDtypeStruct((NP, 128), f32),
            jax.ShapeDtypeStruct((NP, 128), f32),
            jax.ShapeDtypeStruct((H1, HID), f32),
        ],
    )(x_p, W1, as1, ad1)


B2BLK = 2048


def _tc2_body(acc_ref, den_ref, b1_ref, w2_ref, as2_ref, ad2_ref,
              h2_ref, asrc_ref, adst_ref, ms_ref):
    den = den_ref[0][:, 0:8] + den_ref[1][:, 0:8]
    hcat = acc_ref[...].transpose(1, 0, 2)
    o = hcat / den[:, :, None] + b1_ref[...][None]
    o = jnp.where(o > 0, o, jnp.exp(jnp.minimum(o, 0.0)) - 1.0)
    hl = o.reshape(B2BLK, H1 * HID)
    h2 = jnp.dot(hl, w2_ref[...], preferred_element_type=f32)
    a_s = jnp.sum(h2 * as2_ref[...], axis=-1, keepdims=True)
    a_d = jnp.sum(h2 * ad2_ref[...], axis=-1, keepdims=True)
    h2_ref[...] = h2
    asrc_ref[...] = jnp.broadcast_to(a_s, (B2BLK, 128))
    adst_ref[...] = jnp.broadcast_to(a_d, (B2BLK, 128))
    m = jnp.broadcast_to(jnp.max(a_s), (H1, HID))
    @pl.when(pl.program_id(0) == 0)
    def _():
        ms_ref[...] = m
    @pl.when(pl.program_id(0) > 0)
    def _():
        ms_ref[...] = jnp.maximum(ms_ref[...], m)


def _tc2(acc1, denp1, b1r, W2, as2, ad2):
    return pl.pallas_call(
        _tc2_body,
        grid=(NP // B2BLK,),
        in_specs=[
            pl.BlockSpec((H1, B2BLK, HID), lambda i: (0, i, 0)),
            pl.BlockSpec((2, B2BLK, 128), lambda i: (0, i, 0)),
            pl.BlockSpec((H1, HID), lambda i: (0, 0)),
            pl.BlockSpec((H1 * HID, HID), lambda i: (0, 0)),
            pl.BlockSpec((1, HID), lambda i: (0, 0)),
            pl.BlockSpec((1, HID), lambda i: (0, 0)),
        ],
        out_specs=[
            pl.BlockSpec((B2BLK, HID), lambda i: (i, 0)),
            pl.BlockSpec((B2BLK, 128), lambda i: (i, 0)),
            pl.BlockSpec((B2BLK, 128), lambda i: (i, 0)),
            pl.BlockSpec((H1, HID), lambda i: (0, 0)),
        ],
        out_shape=[
            jax.ShapeDtypeStruct((NP, HID), f32),
            jax.ShapeDtypeStruct((NP, 128), f32),
            jax.ShapeDtypeStruct((NP, 128), f32),
            jax.ShapeDtypeStruct((H1, HID), f32),
        ],
    )(acc1, denp1, b1r, W2, as2, ad2)


B3BLK = 2000


def _tc3_body(acc_ref, den_ref, b2_ref, out_ref):
    den = den_ref[0][:, 0:1] + den_ref[1][:, 0:1]
    out_ref[...] = (acc_ref[0] + acc_ref[1]) / den + b2_ref[...]


def _tc3(acc2, denp2, b2r):
    return pl.pallas_call(
        _tc3_body,
        grid=(N // B3BLK,),
        in_specs=[
            pl.BlockSpec((2, B3BLK, HID), lambda i: (0, i, 0)),
            pl.BlockSpec((2, B3BLK, 128), lambda i: (0, i, 0)),
            pl.BlockSpec((1, HID), lambda i: (0, 0)),
        ],
        out_specs=pl.BlockSpec((B3BLK, HID), lambda i: (i, 0)),
        out_shape=jax.ShapeDtypeStruct((N, HID), f32),
    )(acc2, denp2, b2r)


def kernel(x, edge_index, W1, att_src1, att_dst1, b1,
           W2, att_src2, att_dst2, b2):
    pad = jnp.full((EP - E,), NP - 1, dtype=i32)
    src = jnp.concatenate([edge_index[0].astype(i32), pad])
    dst = jnp.concatenate([edge_index[1].astype(i32), pad])
    idx1 = (jnp.arange(H1, dtype=i32)[:, None] * NP
            + src[None, :]).reshape(-1)
    x_p = jnp.pad(x, ((0, NP - N), (0, 0)))

    h1, asrc1, adst1, ms1 = _tc1(x_p, W1,
                                 att_src1.reshape(H1, HID),
                                 att_dst1.reshape(H1, HID))
    ms16_1 = jnp.tile(ms1[:, 0], 2).reshape(1, L)

    ex1, dinit1 = _sca(asrc1, adst1, ms16_1, src, dst)
    acc1, denp1 = _msg1(h1.reshape(H1 * NP, HID), idx1, ex1, dinit1, dst)

    h2, asrc2, adst2, ms2 = _tc2(
        acc1.reshape(H1, NP, HID),
        denp1.reshape(2, NP, 128),
        b1.reshape(H1, HID), W2,
        att_src2.reshape(1, HID), att_dst2.reshape(1, HID))
    ms16_2 = jnp.broadcast_to(ms2[0, 0], (1, L))

    ex2, dinit2 = _sca(asrc2, adst2, ms16_2, src, dst)
    acc2, denp2 = _msg2(h2, src, ex2, dinit2, dst)

    return _tc3(acc2.reshape(2, NP, HID), denp2.reshape(2, NP, 128),
                b2.reshape(1, HID))


# speedup vs baseline: 6.0855x; 1.0854x over previous
import functools

import jax
import jax.numpy as jnp
from jax import lax
from jax.experimental import pallas as pl
from jax.experimental.pallas import tpu as pltpu
from jax.experimental.pallas import tpu_sc as plsc

N = 10000
E = 320000
F_IN = 128
HID = 128
H1 = 8

NC = 2
NS = 16
L = 16
NW = NC * NS
NP = 10240
RT = 320
PR = NP // 16
EP = 327680
BA = 256
BE = 128
SCH = 64

f32 = jnp.float32
i32 = jnp.int32


def _iota():
    return lax.iota(i32, L)


def _splat(v):
    return jnp.full((L,), v, i32)


def _al8(v):
    return pl.multiple_of(v, 8)


def _sca_body(asp, adp, ms16, src, dst,
              ex, dinit,
              idxs, idxd, gbuf, sa, exo, msb, sem):
    cid = lax.axis_index("c")
    sid = lax.axis_index("s")
    wid = cid * NS + sid

    pltpu.sync_copy(ms16, msb)
    msv = msb[0, pl.ds(0, L)]
    lane = _iota()
    lo = lane < 8

    def edge_ex(w_s, w_d):
        z = w_s + w_d
        alpha = jnp.maximum(z, 0.2 * z)
        kk = jnp.maximum(w_d + msv, 0.0)
        return jnp.exp(alpha - kk)

    rc0 = sid * 2 * RT

    def chunk(c, _):
        rc = _al8(rc0 + 80 * c)
        pltpu.sync_copy(asp.at[pl.ds(rc, 80)], gbuf.at[pl.ds(0, 80)])
        pltpu.sync_copy(adp.at[pl.ds(rc, 80)], gbuf.at[pl.ds(80, 80)])

        def selfb(j, _):
            v0 = edge_ex(gbuf[2 * j, pl.ds(0, L)],
                         gbuf[80 + 2 * j, pl.ds(0, L)])
            v1 = edge_ex(gbuf[2 * j + 1, pl.ds(0, L)],
                         gbuf[80 + 2 * j + 1, pl.ds(0, L)])
            exo[j, pl.ds(0, L)] = jnp.where(lo, v0, v1)
            return 0
        lax.fori_loop(0, 40, selfb, 0)
        pltpu.sync_copy(exo.at[pl.ds(0, 40)],
                        dinit.at[pl.ds(_al8(sid * 320 + 40 * c), 40)])
        return 0
    lax.fori_loop(cid * 4, cid * 4 + 4, chunk, 0)

    ebase = wid * (EP // NW)

    def batch(b, _):
        base = _al8(ebase + b * BA)
        pltpu.sync_copy(src.at[pl.ds(base, BA)], idxs)
        pltpu.sync_copy(dst.at[pl.ds(base, BA)], idxd)

        pltpu.async_copy(asp.at[idxs], gbuf, sem).wait()

        def extract(j, _):
            w0 = gbuf[2 * j, pl.ds(0, L)]
            w1 = gbuf[2 * j + 1, pl.ds(0, L)]
            sa[j, pl.ds(0, L)] = jnp.where(lo, w0, w1)
            return 0
        lax.fori_loop(0, BA // 2, extract, 0)

        pltpu.async_copy(adp.at[idxd], gbuf, sem).wait()

        def compute(j, _):
            w0 = gbuf[2 * j, pl.ds(0, L)]
            w1 = gbuf[2 * j + 1, pl.ds(0, L)]
            d = jnp.where(lo, w0, w1)
            exo[j, pl.ds(0, L)] = edge_ex(sa[j, pl.ds(0, L)], d)
            return 0
        lax.fori_loop(0, BA // 2, compute, 0)

        pltpu.sync_copy(exo, ex.at[pl.ds(_al8(base // 2), BA * 8 // L)])
        return 0

    lax.fori_loop(0, EP // NW // BA, batch, 0)


_sca = functools.partial(
    pl.kernel,
    _sca_body,
    out_type=(
        jax.ShapeDtypeStruct((EP * 8 // L, L), f32),
        jax.ShapeDtypeStruct((NP * 8 // L, L), f32),
    ),
    mesh=plsc.VectorSubcoreMesh(core_axis_name="c", subcore_axis_name="s"),
    scratch_types=[
        pltpu.VMEM((BA,), i32),
        pltpu.VMEM((BA,), i32),
        pltpu.VMEM((BA, 128), f32),
        pltpu.VMEM((BA * 8 // L, L), f32),
        pltpu.VMEM((BA * 8 // L, L), f32),
        pltpu.VMEM((1, L), f32),
        pltpu.SemaphoreType.DMA,
    ],
)()


def _row_scale(buf, r, s):
    for k in range(8):
        v = buf[r, pl.ds(k * L, L)]
        buf[r, pl.ds(k * L, L)] = v * s


def _make_msg(nht):
    hps = 4 if nht == 8 else 1
    et = EP // NS if nht == 8 else EP // NW
    nb = et // BE
    out_rows = 8 * NP if nht == 8 else 2 * NP

    def body(table, idxarr, exarr, dinit, dst, out, denomp,
             srcb, dstb, exb, rbuf, dbuf, acc, sem):
        cid = lax.axis_index("c")
        sid = lax.axis_index("s")
        wid = cid * NS + sid
        lane = _iota()
        lane8 = lane % 8
        lane8p = lane8 + 8
        ebase = sid * et if nht == 8 else wid * et
        det = EP // NW
        dbase = wid * det
        dnb = det // BE
        fb = cid * 4
        zb = (1 - cid) * 4

        def acc_range(fill, c0, c1):
            def chunk(c, _):
                rc = _al8(sid * 2 * RT + c * 80)
                fill(c, rc)
                pltpu.sync_copy(rbuf.at[pl.ds(0, 80)], acc.at[pl.ds(rc, 80)])
                return 0
            lax.fori_loop(c0, c1, chunk, 0)

        def zero_rbuf():
            def zro(r, _):
                zv = jnp.zeros((L,), f32)
                for kk in range(8):
                    rbuf[r, pl.ds(kk * L, L)] = zv
                return 0
            lax.fori_loop(0, 80, zro, 0)

        def nofill(c, rc):
            pass

        def writeback(dest, obase):
            def wb(c, _):
                rc = _al8(sid * 2 * RT + c * 80)
                pltpu.sync_copy(acc.at[pl.ds(rc, 80)], rbuf.at[pl.ds(0, 80)])
                pltpu.sync_copy(rbuf.at[pl.ds(0, 80)],
                                dest.at[pl.ds(_al8(obase + rc), 80)])
                return 0
            lax.fori_loop(0, 8, wb, 0)

        for k in range(hps):
            hidx = cid * hps + k if nht == 8 else 0

            def fill_msg(c, rc):
                pltpu.sync_copy(table.at[pl.ds(_al8(hidx * NP + rc), 80)],
                                rbuf.at[pl.ds(0, 80)])
                pltpu.sync_copy(dinit.at[pl.ds(_al8(rc // 2), 40)], dbuf)

                def scale(r, _):
                    fl = r * 8 + hidx
                    w = dbuf[fl // L, pl.ds(0, L)]
                    s = w[jnp.full((L,), fl % L, i32)]
                    _row_scale(rbuf, r, s)
                    return 0
                lax.fori_loop(0, 80, scale, 0)

            if nht == 8:
                acc_range(fill_msg, 0, 8)
            else:
                zero_rbuf()
                acc_range(nofill, zb, zb + 4)
                acc_range(fill_msg, fb, fb + 4)

            plsc.subcore_barrier()

            def batch(b, _):
                base = _al8(ebase + b * BE)
                pltpu.sync_copy(idxarr.at[pl.ds(_al8(hidx * EP + base), BE)],
                                srcb)
                pltpu.sync_copy(exarr.at[pl.ds(_al8(base // 2), BE * 8 // L)],
                                exb)
                pltpu.async_copy(table.at[srcb], rbuf, sem).wait()

                def scale(e, _):
                    fl = e * 8 + hidx
                    w = exb[fl // L, pl.ds(0, L)]
                    s = w[jnp.full((L,), fl % L, i32)]
                    _row_scale(rbuf, e, s)
                    return 0
                lax.fori_loop(0, BE, scale, 0)

                def scat(k5, _):
                    pltpu.sync_copy(dst.at[pl.ds(_al8(base + k5 * SCH), SCH)],
                                    dstb)
                    pltpu.sync_copy(rbuf.at[pl.ds(_al8(k5 * SCH), SCH)],
                                    acc.at[dstb], add=True)
                    return 0
                lax.fori_loop(0, BE // SCH, scat, 0)
                return 0

            lax.fori_loop(0, nb, batch, 0)
            plsc.subcore_barrier()
            writeback(out, hidx * NP if nht == 8 else cid * NP)
            plsc.subcore_barrier()

        def fill_den(c, rc):
            pltpu.sync_copy(dinit.at[pl.ds(_al8(rc // 2), 40)], dbuf)

            def bld(j, _):
                w = dbuf[j, pl.ds(0, L)]
                r0 = w[lane8]
                r1 = w[lane8p]
                for kk in range(8):
                    rbuf[2 * j, pl.ds(kk * L, L)] = r0
                    rbuf[2 * j + 1, pl.ds(kk * L, L)] = r1
                return 0
            lax.fori_loop(0, 40, bld, 0)

        zero_rbuf()
        acc_range(nofill, zb, zb + 4)
        acc_range(fill_den, fb, fb + 4)

        plsc.subcore_barrier()

        def dbatch(b, _):
            base = _al8(dbase + b * BE)
            pltpu.sync_copy(exarr.at[pl.ds(_al8(base // 2), BE * 8 // L)],
                            exb)

            def bld(j, _):
                v = exb[j, pl.ds(0, L)]
                r0 = v[lane8]
                r1 = v[lane8p]
                for kk in range(8):
                    rbuf[2 * j, pl.ds(kk * L, L)] = r0
                    rbuf[2 * j + 1, pl.ds(kk * L, L)] = r1
                return 0
            lax.fori_loop(0, BE // 2, bld, 0)

            def scat(k5, _):
                pltpu.sync_copy(dst.at[pl.ds(_al8(base + k5 * SCH), SCH)],
                                dstb)
                pltpu.sync_copy(rbuf.at[pl.ds(_al8(k5 * SCH), SCH)],
                                acc.at[dstb], add=True)
                return 0
            lax.fori_loop(0, BE // SCH, scat, 0)
            return 0

        lax.fori_loop(0, dnb, dbatch, 0)
        plsc.subcore_barrier()
        writeback(denomp, cid * NP)

    return functools.partial(
        pl.kernel,
        body,
        out_type=(
            jax.ShapeDtypeStruct((out_rows, 128), f32),
            jax.ShapeDtypeStruct((2 * NP, 128), f32),
        ),
        mesh=plsc.VectorSubcoreMesh(core_axis_name="c",
                                    subcore_axis_name="s"),
        scratch_types=[
            pltpu.VMEM((BE,), i32),
            pltpu.VMEM((SCH,), i32),
            pltpu.VMEM((BE * 8 // L, L), f32),
            pltpu.VMEM((BE, 128), f32),
            pltpu.VMEM((40, L), f32),
            pltpu.VMEM_SHARED((NP, 128), f32),
            pltpu.SemaphoreType.DMA,
        ],
    )()


_msg1 = _make_msg(8)
_msg2 = _make_msg(1)


B1BLK = 1024


def _tc1_body(x_ref, w_ref, as_ref, ad_ref, h_ref, asrc_ref, adst_ref,
              ms_ref):
    h = jnp.dot(x_ref[...], w_ref[...], preferred_element_type=f32)
    h3 = h.reshape(B1BLK, H1, HID)
    a_s = jnp.sum(h3 * as_ref[...][None], axis=-1)
    a_d = jnp.sum(h3 * ad_ref[...][None], axis=-1)
    h_ref[...] = h3.transpose(1, 0, 2)
    asrc_ref[...] = jnp.broadcast_to(a_s[:, None, :],
                                     (B1BLK, 16, H1)).reshape(B1BLK, 128)
    adst_ref[...] = jnp.broadcast_to(a_d[:, None, :],
                                     (B1BLK, 16, H1)).reshape(B1BLK, 128)
    m = jnp.broadcast_to(jnp.max(a_s, axis=0)[:, None], (H1, HID))
    @pl.when(pl.program_id(0) == 0)
    def _():
        ms_ref[...] = m
    @pl.when(pl.program_id(0) > 0)
    def _():
        ms_ref[...] = jnp.maximum(ms_ref[...], m)


def _tc1(x_p, W1, as1, ad1):
    return pl.pallas_call(
        _tc1_body,
        grid=(NP // B1BLK,),
        in_specs=[
            pl.BlockSpec((B1BLK, F_IN), lambda i: (i, 0)),
            pl.BlockSpec((F_IN, H1 * HID), lambda i: (0, 0)),
            pl.BlockSpec((H1, HID), lambda i: (0, 0)),
            pl.BlockSpec((H1, HID), lambda i: (0, 0)),
        ],
        out_specs=[
            pl.BlockSpec((H1, B1BLK, HID), lambda i: (0, i, 0)),
            pl.BlockSpec((B1BLK, 128), lambda i: (i, 0)),
            pl.BlockSpec((B1BLK, 128), lambda i: (i, 0)),
            pl.BlockSpec((H1, HID), lambda i: (0, 0)),
        ],
        out_shape=[
            jax.ShapeDtypeStruct((H1, NP, HID), f32),
            jax.ShapeDtypeStruct((NP, 128), f32),
            jax.ShapeDtypeStruct((NP, 128), f32),
            jax.ShapeDtypeStruct((H1, HID), f32),
        ],
    )(x_p, W1, as1, ad1)


B2BLK = 2048


def _tc2_body(acc_ref, den_ref, b1_ref, w2_ref, as2_ref, ad2_ref,
              h2_ref, asrc_ref, adst_ref, ms_ref):
    den = den_ref[0][:, 0:8] + den_ref[1][:, 0:8]
    hcat = acc_ref[...].transpose(1, 0, 2)
    o = hcat / den[:, :, None] + b1_ref[...][None]
    o = jnp.where(o > 0, o, jnp.exp(jnp.minimum(o, 0.0)) - 1.0)
    hl = o.reshape(B2BLK, H1 * HID)
    h2 = jnp.dot(hl, w2_ref[...], preferred_element_type=f32)
    a_s = jnp.sum(h2 * as2_ref[...], axis=-1, keepdims=True)
    a_d = jnp.sum(h2 * ad2_ref[...], axis=-1, keepdims=True)
    h2_ref[...] = h2
    asrc_ref[...] = jnp.broadcast_to(a_s, (B2BLK, 128))
    adst_ref[...] = jnp.broadcast_to(a_d, (B2BLK, 128))
    m = jnp.broadcast_to(jnp.max(a_s), (H1, HID))
    @pl.when(pl.program_id(0) == 0)
    def _():
        ms_ref[...] = m
    @pl.when(pl.program_id(0) > 0)
    def _():
        ms_ref[...] = jnp.maximum(ms_ref[...], m)


def _tc2(acc1, denp1, b1r, W2, as2, ad2):
    return pl.pallas_call(
        _tc2_body,
        grid=(NP // B2BLK,),
        in_specs=[
            pl.BlockSpec((H1, B2BLK, HID), lambda i: (0, i, 0)),
            pl.BlockSpec((2, B2BLK, 128), lambda i: (0, i, 0)),
            pl.BlockSpec((H1, HID), lambda i: (0, 0)),
            pl.BlockSpec((H1 * HID, HID), lambda i: (0, 0)),
            pl.BlockSpec((1, HID), lambda i: (0, 0)),
            pl.BlockSpec((1, HID), lambda i: (0, 0)),
        ],
        out_specs=[
            pl.BlockSpec((B2BLK, HID), lambda i: (i, 0)),
            pl.BlockSpec((B2BLK, 128), lambda i: (i, 0)),
            pl.BlockSpec((B2BLK, 128), lambda i: (i, 0)),
            pl.BlockSpec((H1, HID), lambda i: (0, 0)),
        ],
        out_shape=[
            jax.ShapeDtypeStruct((NP, HID), f32),
            jax.ShapeDtypeStruct((NP, 128), f32),
            jax.ShapeDtypeStruct((NP, 128), f32),
            jax.ShapeDtypeStruct((H1, HID), f32),
        ],
    )(acc1, denp1, b1r, W2, as2, ad2)


B3BLK = 2000


def _tc3_body(acc_ref, den_ref, b2_ref, out_ref):
    den = den_ref[0][:, 0:1] + den_ref[1][:, 0:1]
    out_ref[...] = (acc_ref[0] + acc_ref[1]) / den + b2_ref[...]


def _tc3(acc2, denp2, b2r):
    return pl.pallas_call(
        _tc3_body,
        grid=(N // B3BLK,),
        in_specs=[
            pl.BlockSpec((2, B3BLK, HID), lambda i: (0, i, 0)),
            pl.BlockSpec((2, B3BLK, 128), lambda i: (0, i, 0)),
            pl.BlockSpec((1, HID), lambda i: (0, 0)),
        ],
        out_specs=pl.BlockSpec((B3BLK, HID), lambda i: (i, 0)),
        out_shape=jax.ShapeDtypeStruct((N, HID), f32),
    )(acc2, denp2, b2r)


def kernel(x, edge_index, W1, att_src1, att_dst1, b1,
           W2, att_src2, att_dst2, b2):
    pad = jnp.full((EP - E,), NP - 1, dtype=i32)
    src = jnp.concatenate([edge_index[0].astype(i32), pad])
    dst = jnp.concatenate([edge_index[1].astype(i32), pad])
    idx1 = (jnp.arange(H1, dtype=i32)[:, None] * NP
            + src[None, :]).reshape(-1)
    x_p = jnp.pad(x, ((0, NP - N), (0, 0)))

    h1, asrc1, adst1, ms1 = _tc1(x_p, W1,
                                 att_src1.reshape(H1, HID),
                                 att_dst1.reshape(H1, HID))
    ms16_1 = jnp.tile(ms1[:, 0], 2).reshape(1, L)

    ex1, dinit1 = _sca(asrc1, adst1, ms16_1, src, dst)
    acc1, denp1 = _msg1(h1.reshape(H1 * NP, HID), idx1, ex1, dinit1, dst)

    h2, asrc2, adst2, ms2 = _tc2(
        acc1.reshape(H1, NP, HID),
        denp1.reshape(2, NP, 128),
        b1.reshape(H1, HID), W2,
        att_src2.reshape(1, HID), att_dst2.reshape(1, HID))
    ms16_2 = jnp.broadcast_to(ms2[0, 0], (1, L))

    ex2, dinit2 = _sca(asrc2, adst2, ms16_2, src, dst)
    acc2, denp2 = _msg2(h2, src, ex2, dinit2, dst)

    return _tc3(acc2.reshape(2, NP, HID), denp2.reshape(2, NP, 128),
                b2.reshape(1, HID))


# speedup vs baseline: 6.3471x; 1.0430x over previous
import functools

import jax
import jax.numpy as jnp
from jax import lax
from jax.experimental import pallas as pl
from jax.experimental.pallas import tpu as pltpu
from jax.experimental.pallas import tpu_sc as plsc

N = 10000
E = 320000
F_IN = 128
HID = 128
H1 = 8

NC = 2
NS = 16
L = 16
NW = NC * NS
NP = 10240
RT = 320
PR = NP // 16
EP = 327680
BA = 256
BE = 128
SCH = 128

f32 = jnp.float32
i32 = jnp.int32


def _iota():
    return lax.iota(i32, L)


def _splat(v):
    return jnp.full((L,), v, i32)


def _al8(v):
    return pl.multiple_of(v, 8)


def _sca_body(asp, adp, ms16, src, dst,
              ex, dinit,
              idxs, idxd, gbuf, sa, exo, msb, sem):
    cid = lax.axis_index("c")
    sid = lax.axis_index("s")
    wid = cid * NS + sid

    pltpu.sync_copy(ms16, msb)
    msv = msb[0, pl.ds(0, L)]
    lane = _iota()
    lo = lane < 8

    def edge_ex(w_s, w_d):
        z = w_s + w_d
        alpha = jnp.maximum(z, 0.2 * z)
        kk = jnp.maximum(w_d + msv, 0.0)
        return jnp.exp(alpha - kk)

    rc0 = sid * 2 * RT

    def chunk(c, _):
        rc = _al8(rc0 + 80 * c)
        pltpu.sync_copy(asp.at[pl.ds(rc, 80)], gbuf.at[pl.ds(0, 80)])
        pltpu.sync_copy(adp.at[pl.ds(rc, 80)], gbuf.at[pl.ds(80, 80)])

        def selfb(j, _):
            v0 = edge_ex(gbuf[2 * j, pl.ds(0, L)],
                         gbuf[80 + 2 * j, pl.ds(0, L)])
            v1 = edge_ex(gbuf[2 * j + 1, pl.ds(0, L)],
                         gbuf[80 + 2 * j + 1, pl.ds(0, L)])
            exo[j, pl.ds(0, L)] = jnp.where(lo, v0, v1)
            return 0
        lax.fori_loop(0, 40, selfb, 0)
        pltpu.sync_copy(exo.at[pl.ds(0, 40)],
                        dinit.at[pl.ds(_al8(sid * 320 + 40 * c), 40)])
        return 0
    lax.fori_loop(cid * 4, cid * 4 + 4, chunk, 0)

    ebase = wid * (EP // NW)

    def batch(b, _):
        base = _al8(ebase + b * BA)
        pltpu.sync_copy(src.at[pl.ds(base, BA)], idxs)
        pltpu.sync_copy(dst.at[pl.ds(base, BA)], idxd)

        pltpu.async_copy(asp.at[idxs], gbuf, sem).wait()

        def extract(j, _):
            w0 = gbuf[2 * j, pl.ds(0, L)]
            w1 = gbuf[2 * j + 1, pl.ds(0, L)]
            sa[j, pl.ds(0, L)] = jnp.where(lo, w0, w1)
            return 0
        lax.fori_loop(0, BA // 2, extract, 0)

        pltpu.async_copy(adp.at[idxd], gbuf, sem).wait()

        def compute(j, _):
            w0 = gbuf[2 * j, pl.ds(0, L)]
            w1 = gbuf[2 * j + 1, pl.ds(0, L)]
            d = jnp.where(lo, w0, w1)
            exo[j, pl.ds(0, L)] = edge_ex(sa[j, pl.ds(0, L)], d)
            return 0
        lax.fori_loop(0, BA // 2, compute, 0)

        pltpu.sync_copy(exo, ex.at[pl.ds(_al8(base // 2), BA * 8 // L)])
        return 0

    lax.fori_loop(0, EP // NW // BA, batch, 0)


_sca = functools.partial(
    pl.kernel,
    _sca_body,
    out_type=(
        jax.ShapeDtypeStruct((EP * 8 // L, L), f32),
        jax.ShapeDtypeStruct((NP * 8 // L, L), f32),
    ),
    mesh=plsc.VectorSubcoreMesh(core_axis_name="c", subcore_axis_name="s"),
    scratch_types=[
        pltpu.VMEM((BA,), i32),
        pltpu.VMEM((BA,), i32),
        pltpu.VMEM((BA, 128), f32),
        pltpu.VMEM((BA * 8 // L, L), f32),
        pltpu.VMEM((BA * 8 // L, L), f32),
        pltpu.VMEM((1, L), f32),
        pltpu.SemaphoreType.DMA,
    ],
)()


def _row_scale(buf, r, s):
    for k in range(8):
        v = buf[r, pl.ds(k * L, L)]
        buf[r, pl.ds(k * L, L)] = v * s


def _make_msg(nht):
    hps = 4 if nht == 8 else 1
    et = EP // NS if nht == 8 else EP // NW
    nb = et // BE
    out_rows = 8 * NP if nht == 8 else 2 * NP

    def body(table, idxarr, exarr, dinit, dst, out, denomp,
             srcb, dstb, exb, rbuf, dbuf, acc, sem):
        cid = lax.axis_index("c")
        sid = lax.axis_index("s")
        wid = cid * NS + sid
        lane = _iota()
        lane8 = lane % 8
        lane8p = lane8 + 8
        ebase = sid * et if nht == 8 else wid * et
        det = EP // NW
        dbase = wid * det
        dnb = det // BE
        fb = cid * 4
        zb = (1 - cid) * 4

        def acc_range(fill, c0, c1):
            def chunk(c, _):
                rc = _al8(sid * 2 * RT + c * 80)
                fill(c, rc)
                pltpu.sync_copy(rbuf.at[pl.ds(0, 80)], acc.at[pl.ds(rc, 80)])
                return 0
            lax.fori_loop(c0, c1, chunk, 0)

        def zero_rbuf():
            def zro(r, _):
                zv = jnp.zeros((L,), f32)
                for kk in range(8):
                    rbuf[r, pl.ds(kk * L, L)] = zv
                return 0
            lax.fori_loop(0, 80, zro, 0)

        def nofill(c, rc):
            pass

        def writeback(dest, obase):
            def wb(c, _):
                rc = _al8(sid * 2 * RT + c * 80)
                pltpu.sync_copy(acc.at[pl.ds(rc, 80)], rbuf.at[pl.ds(0, 80)])
                pltpu.sync_copy(rbuf.at[pl.ds(0, 80)],
                                dest.at[pl.ds(_al8(obase + rc), 80)])
                return 0
            lax.fori_loop(0, 8, wb, 0)

        for k in range(hps):
            hidx = cid * hps + k if nht == 8 else 0

            def fill_msg(c, rc):
                pltpu.sync_copy(table.at[pl.ds(_al8(hidx * NP + rc), 80)],
                                rbuf.at[pl.ds(0, 80)])
                pltpu.sync_copy(dinit.at[pl.ds(_al8(rc // 2), 40)], dbuf)

                def scale(r, _):
                    fl = r * 8 + hidx
                    w = dbuf[fl // L, pl.ds(0, L)]
                    s = w[jnp.full((L,), fl % L, i32)]
                    _row_scale(rbuf, r, s)
                    return 0
                lax.fori_loop(0, 80, scale, 0)

            if nht == 8:
                acc_range(fill_msg, 0, 8)
            else:
                zero_rbuf()
                acc_range(nofill, zb, zb + 4)
                acc_range(fill_msg, fb, fb + 4)

            plsc.subcore_barrier()

            def batch(b, _):
                base = _al8(ebase + b * BE)
                pltpu.sync_copy(idxarr.at[pl.ds(_al8(hidx * EP + base), BE)],
                                srcb)
                pltpu.sync_copy(exarr.at[pl.ds(_al8(base // 2), BE * 8 // L)],
                                exb)
                pltpu.async_copy(table.at[srcb], rbuf, sem).wait()

                def scale(e, _):
                    fl = e * 8 + hidx
                    w = exb[fl // L, pl.ds(0, L)]
                    s = w[jnp.full((L,), fl % L, i32)]
                    _row_scale(rbuf, e, s)
                    return 0
                lax.fori_loop(0, BE, scale, 0)

                def scat(k5, _):
                    pltpu.sync_copy(dst.at[pl.ds(_al8(base + k5 * SCH), SCH)],
                                    dstb)
                    pltpu.sync_copy(rbuf.at[pl.ds(_al8(k5 * SCH), SCH)],
                                    acc.at[dstb], add=True)
                    return 0
                lax.fori_loop(0, BE // SCH, scat, 0)
                return 0

            lax.fori_loop(0, nb, batch, 0)
            plsc.subcore_barrier()
            writeback(out, hidx * NP if nht == 8 else cid * NP)
            plsc.subcore_barrier()

        def fill_den(c, rc):
            pltpu.sync_copy(dinit.at[pl.ds(_al8(rc // 2), 40)], dbuf)

            def bld(j, _):
                w = dbuf[j, pl.ds(0, L)]
                r0 = w[lane8]
                r1 = w[lane8p]
                for kk in range(8):
                    rbuf[2 * j, pl.ds(kk * L, L)] = r0
                    rbuf[2 * j + 1, pl.ds(kk * L, L)] = r1
                return 0
            lax.fori_loop(0, 40, bld, 0)

        zero_rbuf()
        acc_range(nofill, zb, zb + 4)
        acc_range(fill_den, fb, fb + 4)

        plsc.subcore_barrier()

        def dbatch(b, _):
            base = _al8(dbase + b * BE)
            pltpu.sync_copy(exarr.at[pl.ds(_al8(base // 2), BE * 8 // L)],
                            exb)

            def bld(j, _):
                v = exb[j, pl.ds(0, L)]
                r0 = v[lane8]
                r1 = v[lane8p]
                for kk in range(8):
                    rbuf[2 * j, pl.ds(kk * L, L)] = r0
                    rbuf[2 * j + 1, pl.ds(kk * L, L)] = r1
                return 0
            lax.fori_loop(0, BE // 2, bld, 0)

            def scat(k5, _):
                pltpu.sync_copy(dst.at[pl.ds(_al8(base + k5 * SCH), SCH)],
                                dstb)
                pltpu.sync_copy(rbuf.at[pl.ds(_al8(k5 * SCH), SCH)],
                                acc.at[dstb], add=True)
                return 0
            lax.fori_loop(0, BE // SCH, scat, 0)
            return 0

        lax.fori_loop(0, dnb, dbatch, 0)
        plsc.subcore_barrier()
        writeback(denomp, cid * NP)

    return functools.partial(
        pl.kernel,
        body,
        out_type=(
            jax.ShapeDtypeStruct((out_rows, 128), f32),
            jax.ShapeDtypeStruct((2 * NP, 128), f32),
        ),
        mesh=plsc.VectorSubcoreMesh(core_axis_name="c",
                                    subcore_axis_name="s"),
        scratch_types=[
            pltpu.VMEM((BE,), i32),
            pltpu.VMEM((SCH,), i32),
            pltpu.VMEM((BE * 8 // L, L), f32),
            pltpu.VMEM((BE, 128), f32),
            pltpu.VMEM((40, L), f32),
            pltpu.VMEM_SHARED((NP, 128), f32),
            pltpu.SemaphoreType.DMA,
        ],
    )()


_msg1 = _make_msg(8)
_msg2 = _make_msg(1)


B1BLK = 1024


def _tc1_body(x_ref, w_ref, as_ref, ad_ref, h_ref, asrc_ref, adst_ref,
              ms_ref):
    h = jnp.dot(x_ref[...], w_ref[...], preferred_element_type=f32)
    h3 = h.reshape(B1BLK, H1, HID)
    a_s = jnp.sum(h3 * as_ref[...][None], axis=-1)
    a_d = jnp.sum(h3 * ad_ref[...][None], axis=-1)
    h_ref[...] = h3.transpose(1, 0, 2)
    asrc_ref[...] = jnp.broadcast_to(a_s[:, None, :],
                                     (B1BLK, 16, H1)).reshape(B1BLK, 128)
    adst_ref[...] = jnp.broadcast_to(a_d[:, None, :],
                                     (B1BLK, 16, H1)).reshape(B1BLK, 128)
    m = jnp.broadcast_to(jnp.max(a_s, axis=0)[:, None], (H1, HID))
    @pl.when(pl.program_id(0) == 0)
    def _():
        ms_ref[...] = m
    @pl.when(pl.program_id(0) > 0)
    def _():
        ms_ref[...] = jnp.maximum(ms_ref[...], m)


def _tc1(x_p, W1, as1, ad1):
    return pl.pallas_call(
        _tc1_body,
        grid=(NP // B1BLK,),
        in_specs=[
            pl.BlockSpec((B1BLK, F_IN), lambda i: (i, 0)),
            pl.BlockSpec((F_IN, H1 * HID), lambda i: (0, 0)),
            pl.BlockSpec((H1, HID), lambda i: (0, 0)),
            pl.BlockSpec((H1, HID), lambda i: (0, 0)),
        ],
        out_specs=[
            pl.BlockSpec((H1, B1BLK, HID), lambda i: (0, i, 0)),
            pl.BlockSpec((B1BLK, 128), lambda i: (i, 0)),
            pl.BlockSpec((B1BLK, 128), lambda i: (i, 0)),
            pl.BlockSpec((H1, HID), lambda i: (0, 0)),
        ],
        out_shape=[
            jax.ShapeDtypeStruct((H1, NP, HID), f32),
            jax.ShapeDtypeStruct((NP, 128), f32),
            jax.ShapeDtypeStruct((NP, 128), f32),
            jax.ShapeDtypeStruct((H1, HID), f32),
        ],
    )(x_p, W1, as1, ad1)


B2BLK = 2048


def _tc2_body(acc_ref, den_ref, b1_ref, w2_ref, as2_ref, ad2_ref,
              h2_ref, asrc_ref, adst_ref, ms_ref):
    den = den_ref[0][:, 0:8] + den_ref[1][:, 0:8]
    hcat = acc_ref[...].transpose(1, 0, 2)
    o = hcat / den[:, :, None] + b1_ref[...][None]
    o = jnp.where(o > 0, o, jnp.exp(jnp.minimum(o, 0.0)) - 1.0)
    hl = o.reshape(B2BLK, H1 * HID)
    h2 = jnp.dot(hl, w2_ref[...], preferred_element_type=f32)
    a_s = jnp.sum(h2 * as2_ref[...], axis=-1, keepdims=True)
    a_d = jnp.sum(h2 * ad2_ref[...], axis=-1, keepdims=True)
    h2_ref[...] = h2
    asrc_ref[...] = jnp.broadcast_to(a_s, (B2BLK, 128))
    adst_ref[...] = jnp.broadcast_to(a_d, (B2BLK, 128))
    m = jnp.broadcast_to(jnp.max(a_s), (H1, HID))
    @pl.when(pl.program_id(0) == 0)
    def _():
        ms_ref[...] = m
    @pl.when(pl.program_id(0) > 0)
    def _():
        ms_ref[...] = jnp.maximum(ms_ref[...], m)


def _tc2(acc1, denp1, b1r, W2, as2, ad2):
    return pl.pallas_call(
        _tc2_body,
        grid=(NP // B2BLK,),
        in_specs=[
            pl.BlockSpec((H1, B2BLK, HID), lambda i: (0, i, 0)),
            pl.BlockSpec((2, B2BLK, 128), lambda i: (0, i, 0)),
            pl.BlockSpec((H1, HID), lambda i: (0, 0)),
            pl.BlockSpec((H1 * HID, HID), lambda i: (0, 0)),
            pl.BlockSpec((1, HID), lambda i: (0, 0)),
            pl.BlockSpec((1, HID), lambda i: (0, 0)),
        ],
        out_specs=[
            pl.BlockSpec((B2BLK, HID), lambda i: (i, 0)),
            pl.BlockSpec((B2BLK, 128), lambda i: (i, 0)),
            pl.BlockSpec((B2BLK, 128), lambda i: (i, 0)),
            pl.BlockSpec((H1, HID), lambda i: (0, 0)),
        ],
        out_shape=[
            jax.ShapeDtypeStruct((NP, HID), f32),
            jax.ShapeDtypeStruct((NP, 128), f32),
            jax.ShapeDtypeStruct((NP, 128), f32),
            jax.ShapeDtypeStruct((H1, HID), f32),
        ],
    )(acc1, denp1, b1r, W2, as2, ad2)


B3BLK = 2000


def _tc3_body(acc_ref, den_ref, b2_ref, out_ref):
    den = den_ref[0][:, 0:1] + den_ref[1][:, 0:1]
    out_ref[...] = (acc_ref[0] + acc_ref[1]) / den + b2_ref[...]


def _tc3(acc2, denp2, b2r):
    return pl.pallas_call(
        _tc3_body,
        grid=(N // B3BLK,),
        in_specs=[
            pl.BlockSpec((2, B3BLK, HID), lambda i: (0, i, 0)),
            pl.BlockSpec((2, B3BLK, 128), lambda i: (0, i, 0)),
            pl.BlockSpec((1, HID), lambda i: (0, 0)),
        ],
        out_specs=pl.BlockSpec((B3BLK, HID), lambda i: (i, 0)),
        out_shape=jax.ShapeDtypeStruct((N, HID), f32),
    )(acc2, denp2, b2r)


def kernel(x, edge_index, W1, att_src1, att_dst1, b1,
           W2, att_src2, att_dst2, b2):
    pad = jnp.full((EP - E,), NP - 1, dtype=i32)
    src = jnp.concatenate([edge_index[0].astype(i32), pad])
    dst = jnp.concatenate([edge_index[1].astype(i32), pad])
    idx1 = (jnp.arange(H1, dtype=i32)[:, None] * NP
            + src[None, :]).reshape(-1)
    x_p = jnp.pad(x, ((0, NP - N), (0, 0)))

    h1, asrc1, adst1, ms1 = _tc1(x_p, W1,
                                 att_src1.reshape(H1, HID),
                                 att_dst1.reshape(H1, HID))
    ms16_1 = jnp.tile(ms1[:, 0], 2).reshape(1, L)

    ex1, dinit1 = _sca(asrc1, adst1, ms16_1, src, dst)
    acc1, denp1 = _msg1(h1.reshape(H1 * NP, HID), idx1, ex1, dinit1, dst)

    h2, asrc2, adst2, ms2 = _tc2(
        acc1.reshape(H1, NP, HID),
        denp1.reshape(2, NP, 128),
        b1.reshape(H1, HID), W2,
        att_src2.reshape(1, HID), att_dst2.reshape(1, HID))
    ms16_2 = jnp.broadcast_to(ms2[0, 0], (1, L))

    ex2, dinit2 = _sca(asrc2, adst2, ms16_2, src, dst)
    acc2, denp2 = _msg2(h2, src, ex2, dinit2, dst)

    return _tc3(acc2.reshape(2, NP, HID), denp2.reshape(2, NP, 128),
                b2.reshape(1, HID))
